# baseline (speedup 1.0000x reference)
import functools

import jax
import jax.numpy as jnp
from jax import lax
from jax.experimental import pallas as pl
from jax.experimental.pallas import tpu as pltpu
from jax.experimental.pallas import tpu_sc as plsc

VOCAB = 100000
EMBED = 16
WINDOW = 20
BATCH = 1024

NUM_CORES = 2
NUM_SUBCORES = 16
NW = NUM_CORES * NUM_SUBCORES
B_PER_W = BATCH // NW
ROWS_PER_W = B_PER_W * WINDOW

VBLK = 2048
VGRID = (VOCAB + VBLK - 1) // VBLK


def _rsqrt16(s):
    s = jnp.maximum(s, jnp.float32(1e-12))
    i = lax.bitcast_convert_type(s, jnp.int32)
    i = jnp.int32(0x5F3759DF) - (i >> 1)
    y = lax.bitcast_convert_type(i, jnp.float32)
    for _ in range(3):
        y = y * (jnp.float32(1.5) - jnp.float32(0.5) * s * y * y)
    return y


def _sc_pool_body(idx_hbm, table_hbm, out_hbm, idx_v, rows_v, pool_v, sem):
    wid = lax.axis_index("s") * NUM_CORES + lax.axis_index("c")
    base = wid * ROWS_PER_W
    pltpu.sync_copy(idx_hbm.at[pl.ds(base, ROWS_PER_W)], idx_v)
    pltpu.async_copy(table_hbm.at[idx_v], rows_v, sem).wait()

    inv_w = jnp.float32(1.0 / WINDOW)

    def outer(b, carry):
        def inner(w, acc):
            r = b * WINDOW + w
            v = rows_v[r]
            ss = jnp.broadcast_to(jnp.sum(v * v), (16,))
            rs = _rsqrt16(ss)
            norm = ss * rs
            scale = jnp.where(ss > jnp.float32(1.0),
                              jnp.float32(1.0) / (norm + jnp.float32(1e-7)),
                              jnp.float32(1.0))
            return acc + v * scale

        acc = lax.fori_loop(0, WINDOW, inner, jnp.zeros((16,), jnp.float32))
        pool_v[b] = acc * inv_w
        return carry

    lax.fori_loop(0, B_PER_W, outer, 0)
    pltpu.sync_copy(pool_v, out_hbm.at[pl.ds(wid * B_PER_W, B_PER_W)])


@functools.cache
def _sc_pool():
    return pl.kernel(
        _sc_pool_body,
        mesh=plsc.VectorSubcoreMesh(core_axis_name="c", subcore_axis_name="s"),
        out_type=jax.ShapeDtypeStruct((BATCH, EMBED), jnp.float32),
        scratch_types=[
            pltpu.VMEM((ROWS_PER_W,), jnp.int32),
            pltpu.VMEM((ROWS_PER_W, EMBED), jnp.float32),
            pltpu.VMEM((B_PER_W, EMBED), jnp.float32),
            pltpu.SemaphoreType.DMA,
        ],
        compiler_params=pltpu.CompilerParams(
            needs_layout_passes=False, use_tc_tiling_on_sc=False
        ),
    )


def _mm_body(p_ref, w_ref, b_ref, o_ref):
    o_ref[...] = lax.dot_general(
        p_ref[...], w_ref[...],
        dimension_numbers=(((1,), (1,)), ((), ())),
        preferred_element_type=jnp.float32,
    ) + b_ref[...]


def _tc_project(pooled, W, b2):
    return pl.pallas_call(
        _mm_body,
        grid=(VGRID,),
        in_specs=[
            pl.BlockSpec((BATCH, EMBED), lambda j: (0, 0)),
            pl.BlockSpec((VBLK, EMBED), lambda j: (j, 0)),
            pl.BlockSpec((1, VBLK), lambda j: (0, j)),
        ],
        out_specs=pl.BlockSpec((BATCH, VBLK), lambda j: (0, j)),
        out_shape=jax.ShapeDtypeStruct((BATCH, VOCAB), jnp.float32),
        compiler_params=pltpu.CompilerParams(
            dimension_semantics=("parallel",),
        ),
    )(pooled, W, b2)


def kernel(x, table, W, b):
    idx = x.reshape(-1).astype(jnp.int32)
    pooled = _sc_pool()(idx, table)
    return _tc_project(pooled, W, b.reshape(1, VOCAB))
